# Initial kernel scaffold; baseline (speedup 1.0000x reference)
import jax
import jax.numpy as jnp
from jax.experimental import pallas as pl


def kernel(x, edge_index, edge_attr, W_h, b_h, W_l, b_l, W_g, att_src, att_dst, g_in, b_in, g_n, b_n, b_param):
    raise NotImplementedError("write your pallas kernel here")


# speedup vs baseline: 28.7944x; 28.7944x over previous
import functools

import jax
import jax.numpy as jnp
from jax import lax
from jax.experimental import pallas as pl
from jax.experimental.pallas import tpu as pltpu
from jax.experimental.pallas import tpu_sc as plsc

_N = 10000
_E = 320000
_D = 128
_H = 8
_DH = 16
_EPS = 1e-5
_NEG = 0.2
_INV_LN2 = 1.4426950408889634

_NC = 2
_NS = 16
_NW = _NC * _NS
_K = 64
_CPT = 158
_EP = _NW * _CPT * _K
_NP = 10240
_ROWS = _NP // _NS

_BN = 1000


def _pre_body(x_ref, wh_ref, bh_ref, wl_ref, bl_ref, wg_ref, ms_ref, md_ref,
              gin_ref, bin_ref, x0_ref, h_ref, xl_ref, xp_ref, a1_ref, a2_ref):
    x = x_ref[...]
    mu = jnp.mean(x, axis=1, keepdims=True)
    xc = x - mu
    var = jnp.mean(xc * xc, axis=1, keepdims=True)
    x0 = xc * lax.rsqrt(var + _EPS) * gin_ref[...] + bin_ref[...]
    x0_ref[...] = x0
    h_ref[...] = jnp.maximum(
        jnp.dot(x0, wh_ref[...], preferred_element_type=jnp.float32) + bh_ref[...], 0.0)
    xl_ref[...] = jnp.dot(x0, wl_ref[...], preferred_element_type=jnp.float32) + bl_ref[...]
    xp = jnp.dot(x0, wg_ref[...], preferred_element_type=jnp.float32)
    xp_ref[...] = xp
    a_s = jnp.dot(xp, ms_ref[...], preferred_element_type=jnp.float32)
    a_d = jnp.dot(xp, md_ref[...], preferred_element_type=jnp.float32)
    a1_ref[...] = jnp.concatenate([a_s, jnp.zeros_like(a_d)], axis=1)
    a2_ref[...] = jnp.concatenate([a_d, jnp.zeros_like(a_s)], axis=1)


def _wexp_body(w_ref, o_ref):
    o_ref[...] = jnp.exp(jnp.log(w_ref[...]) * _INV_LN2)


def _post_body(a0_ref, a1_ref, d0_ref, d1_ref, x0_ref, h_ref, xl_ref, r_ref,
               gn_ref, bn_ref, beta_ref, out_ref):
    den = d0_ref[...] + d1_ref[...] + 1e-16
    r128 = jnp.dot(1.0 / den, r_ref[...], preferred_element_type=jnp.float32)
    gat = (a0_ref[...] + a1_ref[...]) * r128
    xg = jnp.maximum(gat + xl_ref[...], 0.0)
    m = h_ref[...] * xg
    mu = jnp.mean(m, axis=1, keepdims=True)
    mc = m - mu
    var = jnp.mean(mc * mc, axis=1, keepdims=True)
    ln = mc * lax.rsqrt(var + _EPS) * gn_ref[...] + bn_ref[...]
    beta = beta_ref[...]
    out_ref[...] = (1.0 - beta) * ln + beta * xg + x0_ref[...]


def _sc_edge_body(src_hbm, dst_hbm, w_hbm, a1_hbm, a2_hbm, xp_hbm, z128_hbm,
                  z16_hbm, acc_hbm, den_hbm,
                  src_v, dst_v, w_v, as_v, ad_v, e_v, xp_v, msg_v,
                  acc_sh, den_sh, sem1, sem2, sem3):
    cid = lax.axis_index("c")
    sid = lax.axis_index("s")
    wid = cid * _NS + sid

    base = sid * _ROWS
    pltpu.sync_copy(z128_hbm.at[pl.ds(base, _ROWS)], acc_sh.at[pl.ds(base, _ROWS)])
    pltpu.sync_copy(z16_hbm.at[pl.ds(base, _ROWS)], den_sh.at[pl.ds(base, _ROWS)])
    plsc.subcore_barrier()

    def chunk_body(c, _):
        row = wid * _CPT + c
        pltpu.sync_copy(src_hbm.at[row], src_v)
        pltpu.sync_copy(dst_hbm.at[row], dst_v)
        pltpu.sync_copy(w_hbm.at[row], w_v)
        cp1 = pltpu.async_copy(a1_hbm.at[src_v], as_v, sem1)
        cp2 = pltpu.async_copy(a2_hbm.at[dst_v], ad_v, sem2)
        cp3 = pltpu.async_copy(xp_hbm.at[src_v], xp_v, sem3)
        cp1.wait()
        cp2.wait()
        cp3.wait()

        def edge_body(k, _):
            z = as_v[k, :] + ad_v[k, :]
            lr = jnp.where(z > 0.0, z, z * _NEG)
            e = jnp.exp(lr) * w_v[k, :]
            e_v[k, :] = e
            for h in range(_H):
                mult = jnp.full((16,), e[h], jnp.float32)
                msg_v[k, pl.ds(h * 16, 16)] = xp_v[k, pl.ds(h * 16, 16)] * mult
            return 0

        lax.fori_loop(0, _K, edge_body, 0)
        pltpu.sync_copy(e_v, den_sh.at[dst_v], add=True)
        pltpu.sync_copy(msg_v, acc_sh.at[dst_v], add=True)
        return 0

    lax.fori_loop(0, _CPT, chunk_body, 0)
    plsc.subcore_barrier()

    pltpu.sync_copy(acc_sh.at[pl.ds(base, _ROWS)], acc_hbm.at[cid, pl.ds(base, _ROWS)])
    pltpu.sync_copy(den_sh.at[pl.ds(base, _ROWS)], den_hbm.at[cid, pl.ds(base, _ROWS)])


_sc_edge = functools.partial(
    pl.kernel,
    out_type=(jax.ShapeDtypeStruct((_NC, _NP, _D), jnp.float32),
              jax.ShapeDtypeStruct((_NC, _NP, 2 * _H), jnp.float32)),
    mesh=plsc.VectorSubcoreMesh(core_axis_name="c", subcore_axis_name="s"),
    scratch_types=[
        pltpu.VMEM((_K,), jnp.int32),
        pltpu.VMEM((_K,), jnp.int32),
        pltpu.VMEM((_K, 16), jnp.float32),
        pltpu.VMEM((_K, 2 * _H), jnp.float32),
        pltpu.VMEM((_K, 2 * _H), jnp.float32),
        pltpu.VMEM((_K, 2 * _H), jnp.float32),
        pltpu.VMEM((_K, _D), jnp.float32),
        pltpu.VMEM((_K, _D), jnp.float32),
        pltpu.VMEM_SHARED((_NP, _D), jnp.float32),
        pltpu.VMEM_SHARED((_NP, 2 * _H), jnp.float32),
        pltpu.SemaphoreType.DMA,
        pltpu.SemaphoreType.DMA,
        pltpu.SemaphoreType.DMA,
    ],
    compiler_params=pltpu.CompilerParams(use_tc_tiling_on_sc=False,
                                         needs_layout_passes=False),
)(_sc_edge_body)


def kernel(x, edge_index, edge_attr, W_h, b_h, W_l, b_l, W_g, att_src, att_dst,
           g_in, b_in, g_n, b_n, b_param):
    rows = jnp.arange(_D, dtype=jnp.int32)
    ms = jnp.zeros((_D, _H), jnp.float32).at[rows, rows // _DH].set(att_src.reshape(-1))
    md = jnp.zeros((_D, _H), jnp.float32).at[rows, rows // _DH].set(att_dst.reshape(-1))
    rexp = jnp.zeros((_H, _D), jnp.float32).at[rows // _DH, rows].set(1.0)

    grid = _N // _BN
    row_spec = pl.BlockSpec((_BN, _D), lambda i: (i, 0))
    w_spec = pl.BlockSpec((_D, _D), lambda i: (0, 0))
    b_spec = pl.BlockSpec((1, _D), lambda i: (0, 0))
    m_spec = pl.BlockSpec((_D, _H), lambda i: (0, 0))
    a_spec = pl.BlockSpec((_BN, 2 * _H), lambda i: (i, 0))

    x0, h, xl, xp, atab1, atab2 = pl.pallas_call(
        _pre_body,
        grid=(grid,),
        in_specs=[row_spec, w_spec, b_spec, w_spec, b_spec, w_spec, m_spec,
                  m_spec, b_spec, b_spec],
        out_specs=[row_spec, row_spec, row_spec, row_spec, a_spec, a_spec],
        out_shape=[jax.ShapeDtypeStruct((_N, _D), jnp.float32)] * 4
        + [jax.ShapeDtypeStruct((_N, 2 * _H), jnp.float32)] * 2,
    )(x, W_h, b_h.reshape(1, _D), W_l, b_l.reshape(1, _D), W_g, ms, md,
      g_in.reshape(1, _D), b_in.reshape(1, _D))

    w2 = pl.pallas_call(
        _wexp_body,
        out_shape=jax.ShapeDtypeStruct((_E // _D, _D), jnp.float32),
    )(edge_attr.reshape(_E // _D, _D))

    pad = _EP - _E
    src = jnp.pad(edge_index[0].astype(jnp.int32), (0, pad)).reshape(_EP // _K, _K)
    dst = jnp.pad(edge_index[1].astype(jnp.int32), (0, pad)).reshape(_EP // _K, _K)
    w2p = jnp.broadcast_to(
        jnp.pad(w2.reshape(-1), (0, pad)).reshape(_EP // _K, _K, 1),
        (_EP // _K, _K, 16))
    z128 = jnp.zeros((_NP, _D), jnp.float32)
    z16 = jnp.zeros((_NP, 2 * _H), jnp.float32)

    acc, den = _sc_edge(src, dst, w2p, atab1, atab2, xp, z128, z16)

    d0 = den[0, :_N, :_H]
    d1 = den[1, :_N, :_H]
    d_spec = pl.BlockSpec((_BN, _H), lambda i: (i, 0))
    r_spec = pl.BlockSpec((_H, _D), lambda i: (0, 0))
    out = pl.pallas_call(
        _post_body,
        grid=(grid,),
        in_specs=[row_spec, row_spec, d_spec, d_spec, row_spec, row_spec,
                  row_spec, r_spec, b_spec, b_spec, b_spec],
        out_specs=row_spec,
        out_shape=jax.ShapeDtypeStruct((_N, _D), jnp.float32),
    )(acc[0, :_N], acc[1, :_N], d0, d1, x0, h, xl, rexp,
      g_n.reshape(1, _D), b_n.reshape(1, _D), b_param.reshape(1, _D))
    return out


# speedup vs baseline: 28.9441x; 1.0052x over previous
import functools

import jax
import jax.numpy as jnp
from jax import lax
from jax.experimental import pallas as pl
from jax.experimental.pallas import tpu as pltpu
from jax.experimental.pallas import tpu_sc as plsc

_N = 10000
_E = 320000
_D = 128
_H = 8
_DH = 16
_EPS = 1e-5
_NEG = 0.2
_INV_LN2 = 1.4426950408889634

_NC = 2
_NS = 16
_NW = _NC * _NS
_K = 64
_CPT = 158
_EP = _NW * _CPT * _K
_NP = 10240
_ROWS = _NP // _NS

_BN = 1000


def _pre_body(x_ref, wh_ref, bh_ref, wl_ref, bl_ref, wg_ref, ms_ref, md_ref,
              gin_ref, bin_ref, x0_ref, h_ref, xl_ref, xp_ref, a1_ref, a2_ref):
    x = x_ref[...]
    mu = jnp.mean(x, axis=1, keepdims=True)
    xc = x - mu
    var = jnp.mean(xc * xc, axis=1, keepdims=True)
    x0 = xc * lax.rsqrt(var + _EPS) * gin_ref[...] + bin_ref[...]
    x0_ref[...] = x0
    h_ref[...] = jnp.maximum(
        jnp.dot(x0, wh_ref[...], preferred_element_type=jnp.float32) + bh_ref[...], 0.0)
    xl_ref[...] = jnp.dot(x0, wl_ref[...], preferred_element_type=jnp.float32) + bl_ref[...]
    xp = jnp.dot(x0, wg_ref[...], preferred_element_type=jnp.float32)
    xp_ref[...] = xp
    a_s = jnp.dot(xp, ms_ref[...], preferred_element_type=jnp.float32)
    a_d = jnp.dot(xp, md_ref[...], preferred_element_type=jnp.float32)
    a1_ref[...] = jnp.concatenate([a_s, jnp.zeros_like(a_d)], axis=1)
    a2_ref[...] = jnp.concatenate([a_d, jnp.zeros_like(a_s)], axis=1)


def _wexp_body(w_ref, o_ref):
    o_ref[...] = jnp.exp(jnp.log(w_ref[...]) * _INV_LN2)


def _post_body(a0_ref, a1_ref, d0_ref, d1_ref, x0_ref, h_ref, xl_ref, r_ref,
               gn_ref, bn_ref, beta_ref, out_ref):
    den = d0_ref[...] + d1_ref[...] + 1e-16
    r128 = jnp.dot(1.0 / den, r_ref[...], preferred_element_type=jnp.float32)
    gat = (a0_ref[...] + a1_ref[...]) * r128
    xg = jnp.maximum(gat + xl_ref[...], 0.0)
    m = h_ref[...] * xg
    mu = jnp.mean(m, axis=1, keepdims=True)
    mc = m - mu
    var = jnp.mean(mc * mc, axis=1, keepdims=True)
    ln = mc * lax.rsqrt(var + _EPS) * gn_ref[...] + bn_ref[...]
    beta = beta_ref[...]
    out_ref[...] = (1.0 - beta) * ln + beta * xg + x0_ref[...]


def _sc_edge_body(src_hbm, dst_hbm, w_hbm, a1_hbm, a2_hbm, xp_hbm, z128_hbm,
                  z16_hbm, acc_hbm, den_hbm,
                  src_v, dst_v, w_v, as_v, ad_v, e_v, xp_v, msg_v,
                  acc_sh, den_sh, sem1, sem2, sem3):
    cid = lax.axis_index("c")
    sid = lax.axis_index("s")
    wid = cid * _NS + sid

    base = sid * _ROWS
    pltpu.sync_copy(z128_hbm.at[pl.ds(base, _ROWS)], acc_sh.at[pl.ds(base, _ROWS)])
    pltpu.sync_copy(z16_hbm.at[pl.ds(base, _ROWS)], den_sh.at[pl.ds(base, _ROWS)])
    plsc.subcore_barrier()

    def chunk_body(c, _):
        row = wid * _CPT + c
        pltpu.sync_copy(src_hbm.at[row], src_v)
        pltpu.sync_copy(dst_hbm.at[row], dst_v)
        pltpu.sync_copy(w_hbm.at[row], w_v)
        cp1 = pltpu.async_copy(a1_hbm.at[src_v], as_v, sem1)
        cp2 = pltpu.async_copy(a2_hbm.at[dst_v], ad_v, sem2)
        cp3 = pltpu.async_copy(xp_hbm.at[src_v], xp_v, sem3)
        cp1.wait()
        cp2.wait()
        cp3.wait()

        def edge_body(k, _):
            z = as_v[k, :] + ad_v[k, :]
            lr = jnp.where(z > 0.0, z, z * _NEG)
            e = jnp.exp(lr) * w_v[k, :]
            e_v[k, :] = e
            for h in range(_H):
                mult = jnp.full((16,), e[h], jnp.float32)
                msg_v[k, pl.ds(h * 16, 16)] = xp_v[k, pl.ds(h * 16, 16)] * mult
            return 0

        lax.fori_loop(0, _K, edge_body, 0, unroll=4)
        pltpu.sync_copy(e_v, den_sh.at[dst_v], add=True)
        pltpu.sync_copy(msg_v, acc_sh.at[dst_v], add=True)
        return 0

    lax.fori_loop(0, _CPT, chunk_body, 0)
    plsc.subcore_barrier()

    pltpu.sync_copy(acc_sh.at[pl.ds(base, _ROWS)], acc_hbm.at[cid, pl.ds(base, _ROWS)])
    pltpu.sync_copy(den_sh.at[pl.ds(base, _ROWS)], den_hbm.at[cid, pl.ds(base, _ROWS)])


_sc_edge = functools.partial(
    pl.kernel,
    out_type=(jax.ShapeDtypeStruct((_NC, _NP, _D), jnp.float32),
              jax.ShapeDtypeStruct((_NC, _NP, 2 * _H), jnp.float32)),
    mesh=plsc.VectorSubcoreMesh(core_axis_name="c", subcore_axis_name="s"),
    scratch_types=[
        pltpu.VMEM((_K,), jnp.int32),
        pltpu.VMEM((_K,), jnp.int32),
        pltpu.VMEM((_K, 16), jnp.float32),
        pltpu.VMEM((_K, 2 * _H), jnp.float32),
        pltpu.VMEM((_K, 2 * _H), jnp.float32),
        pltpu.VMEM((_K, 2 * _H), jnp.float32),
        pltpu.VMEM((_K, _D), jnp.float32),
        pltpu.VMEM((_K, _D), jnp.float32),
        pltpu.VMEM_SHARED((_NP, _D), jnp.float32),
        pltpu.VMEM_SHARED((_NP, 2 * _H), jnp.float32),
        pltpu.SemaphoreType.DMA,
        pltpu.SemaphoreType.DMA,
        pltpu.SemaphoreType.DMA,
    ],
    compiler_params=pltpu.CompilerParams(use_tc_tiling_on_sc=False,
                                         needs_layout_passes=False),
)(_sc_edge_body)


def kernel(x, edge_index, edge_attr, W_h, b_h, W_l, b_l, W_g, att_src, att_dst,
           g_in, b_in, g_n, b_n, b_param):
    rows = jnp.arange(_D, dtype=jnp.int32)
    ms = jnp.zeros((_D, _H), jnp.float32).at[rows, rows // _DH].set(att_src.reshape(-1))
    md = jnp.zeros((_D, _H), jnp.float32).at[rows, rows // _DH].set(att_dst.reshape(-1))
    rexp = jnp.zeros((_H, _D), jnp.float32).at[rows // _DH, rows].set(1.0)

    grid = _N // _BN
    row_spec = pl.BlockSpec((_BN, _D), lambda i: (i, 0))
    w_spec = pl.BlockSpec((_D, _D), lambda i: (0, 0))
    b_spec = pl.BlockSpec((1, _D), lambda i: (0, 0))
    m_spec = pl.BlockSpec((_D, _H), lambda i: (0, 0))
    a_spec = pl.BlockSpec((_BN, 2 * _H), lambda i: (i, 0))

    x0, h, xl, xp, atab1, atab2 = pl.pallas_call(
        _pre_body,
        grid=(grid,),
        in_specs=[row_spec, w_spec, b_spec, w_spec, b_spec, w_spec, m_spec,
                  m_spec, b_spec, b_spec],
        out_specs=[row_spec, row_spec, row_spec, row_spec, a_spec, a_spec],
        out_shape=[jax.ShapeDtypeStruct((_N, _D), jnp.float32)] * 4
        + [jax.ShapeDtypeStruct((_N, 2 * _H), jnp.float32)] * 2,
    )(x, W_h, b_h.reshape(1, _D), W_l, b_l.reshape(1, _D), W_g, ms, md,
      g_in.reshape(1, _D), b_in.reshape(1, _D))

    w2 = pl.pallas_call(
        _wexp_body,
        out_shape=jax.ShapeDtypeStruct((_E // _D, _D), jnp.float32),
    )(edge_attr.reshape(_E // _D, _D))

    pad = _EP - _E
    src = jnp.pad(edge_index[0].astype(jnp.int32), (0, pad)).reshape(_EP // _K, _K)
    dst = jnp.pad(edge_index[1].astype(jnp.int32), (0, pad)).reshape(_EP // _K, _K)
    w2p = jnp.broadcast_to(
        jnp.pad(w2.reshape(-1), (0, pad)).reshape(_EP // _K, _K, 1),
        (_EP // _K, _K, 16))
    z128 = jnp.zeros((_NP, _D), jnp.float32)
    z16 = jnp.zeros((_NP, 2 * _H), jnp.float32)

    acc, den = _sc_edge(src, dst, w2p, atab1, atab2, xp, z128, z16)

    d0 = den[0, :_N, :_H]
    d1 = den[1, :_N, :_H]
    d_spec = pl.BlockSpec((_BN, _H), lambda i: (i, 0))
    r_spec = pl.BlockSpec((_H, _D), lambda i: (0, 0))
    out = pl.pallas_call(
        _post_body,
        grid=(grid,),
        in_specs=[row_spec, row_spec, d_spec, d_spec, row_spec, row_spec,
                  row_spec, r_spec, b_spec, b_spec, b_spec],
        out_specs=row_spec,
        out_shape=jax.ShapeDtypeStruct((_N, _D), jnp.float32),
    )(acc[0, :_N], acc[1, :_N], d0, d1, x0, h, xl, rexp,
      g_n.reshape(1, _D), b_n.reshape(1, _D), b_param.reshape(1, _D))
    return out


# speedup vs baseline: 41.3003x; 1.4269x over previous
import functools

import jax
import jax.numpy as jnp
from jax import lax
from jax.experimental import pallas as pl
from jax.experimental.pallas import tpu as pltpu
from jax.experimental.pallas import tpu_sc as plsc

_N = 10000
_E = 320000
_D = 128
_H = 8
_DH = 16
_EPS = 1e-5
_NEG = 0.2
_INV_LN2 = 1.4426950408889634

_NC = 2
_NS = 16
_NW = _NC * _NS
_K = 64
_CPT = 158
_EP = _NW * _CPT * _K
_NP = 10240
_ROWS = _NP // _NS

_BN = 1000


def _pre_body(x_ref, wh_ref, bh_ref, wl_ref, bl_ref, wg_ref, ms_ref, md_ref,
              gin_ref, bin_ref, x0_ref, h_ref, xl_ref, xp_ref, a1_ref, a2_ref):
    x = x_ref[...]
    mu = jnp.mean(x, axis=1, keepdims=True)
    xc = x - mu
    var = jnp.mean(xc * xc, axis=1, keepdims=True)
    x0 = xc * lax.rsqrt(var + _EPS) * gin_ref[...] + bin_ref[...]
    x0_ref[...] = x0
    h_ref[...] = jnp.maximum(
        jnp.dot(x0, wh_ref[...], preferred_element_type=jnp.float32) + bh_ref[...], 0.0)
    xl_ref[...] = jnp.dot(x0, wl_ref[...], preferred_element_type=jnp.float32) + bl_ref[...]
    xp = jnp.dot(x0, wg_ref[...], preferred_element_type=jnp.float32)
    xp_ref[...] = xp
    a_s = jnp.dot(xp, ms_ref[...], preferred_element_type=jnp.float32)
    a_d = jnp.dot(xp, md_ref[...], preferred_element_type=jnp.float32)
    a1_ref[...] = jnp.concatenate([a_s, jnp.zeros_like(a_d)], axis=1)
    a2_ref[...] = jnp.concatenate([a_d, jnp.zeros_like(a_s)], axis=1)


def _wexp_body(w_ref, o_ref):
    o_ref[...] = jnp.exp(jnp.log(w_ref[...]) * _INV_LN2)


def _post_body(a0_ref, a1_ref, d0_ref, d1_ref, x0_ref, h_ref, xl_ref, r_ref,
               gn_ref, bn_ref, beta_ref, out_ref):
    den = d0_ref[...] + d1_ref[...] + 1e-16
    r128 = jnp.dot(1.0 / den, r_ref[...], preferred_element_type=jnp.float32)
    gat = (a0_ref[...] + a1_ref[...]) * r128
    xg = jnp.maximum(gat + xl_ref[...], 0.0)
    m = h_ref[...] * xg
    mu = jnp.mean(m, axis=1, keepdims=True)
    mc = m - mu
    var = jnp.mean(mc * mc, axis=1, keepdims=True)
    ln = mc * lax.rsqrt(var + _EPS) * gn_ref[...] + bn_ref[...]
    beta = beta_ref[...]
    out_ref[...] = (1.0 - beta) * ln + beta * xg + x0_ref[...]


def _sc_edge_body(src_hbm, dst_hbm, w_hbm, a1_hbm, a2_hbm, xp_hbm, z128_hbm,
                  z16_hbm, acc_hbm, den_hbm,
                  src_v, dst_v, w_v, as_v, ad_v, e_v, xp_v, msg_v,
                  acc_sh, den_sh, sem1, sem2, sem3):
    cid = lax.axis_index("c")
    sid = lax.axis_index("s")
    wid = cid * _NS + sid

    base = sid * _ROWS
    pltpu.sync_copy(z128_hbm.at[pl.ds(base, _ROWS)], acc_sh.at[pl.ds(base, _ROWS)])
    pltpu.sync_copy(z16_hbm.at[pl.ds(base, _ROWS)], den_sh.at[pl.ds(base, _ROWS)])
    plsc.subcore_barrier()

    def chunk_body(c, _):
        row = wid * _CPT + c
        pltpu.sync_copy(src_hbm.at[row], src_v)
        pltpu.sync_copy(dst_hbm.at[row], dst_v)
        pltpu.sync_copy(w_hbm.at[row], w_v)
        cp1 = pltpu.async_copy(a1_hbm.at[src_v], as_v, sem1)
        cp2 = pltpu.async_copy(a2_hbm.at[dst_v], ad_v, sem2)
        cp3 = pltpu.async_copy(xp_hbm.at[src_v], xp_v, sem3)
        cp1.wait()
        cp2.wait()
        cp3.wait()

        @plsc.parallel_loop(0, _K, unroll=4)
        def edge_body(k):
            z = as_v[k, :] + ad_v[k, :]
            lr = jnp.where(z > 0.0, z, z * _NEG)
            e = jnp.exp(lr) * w_v[k, :]
            e_v[k, :] = e
            for h in range(_H):
                mult = jnp.full((16,), e[h], jnp.float32)
                msg_v[k, pl.ds(h * 16, 16)] = xp_v[k, pl.ds(h * 16, 16)] * mult
        pltpu.sync_copy(e_v, den_sh.at[dst_v], add=True)
        pltpu.sync_copy(msg_v, acc_sh.at[dst_v], add=True)
        return 0

    lax.fori_loop(0, _CPT, chunk_body, 0)
    plsc.subcore_barrier()

    pltpu.sync_copy(acc_sh.at[pl.ds(base, _ROWS)], acc_hbm.at[cid, pl.ds(base, _ROWS)])
    pltpu.sync_copy(den_sh.at[pl.ds(base, _ROWS)], den_hbm.at[cid, pl.ds(base, _ROWS)])


_sc_edge = functools.partial(
    pl.kernel,
    out_type=(jax.ShapeDtypeStruct((_NC, _NP, _D), jnp.float32),
              jax.ShapeDtypeStruct((_NC, _NP, 2 * _H), jnp.float32)),
    mesh=plsc.VectorSubcoreMesh(core_axis_name="c", subcore_axis_name="s"),
    scratch_types=[
        pltpu.VMEM((_K,), jnp.int32),
        pltpu.VMEM((_K,), jnp.int32),
        pltpu.VMEM((_K, 16), jnp.float32),
        pltpu.VMEM((_K, 2 * _H), jnp.float32),
        pltpu.VMEM((_K, 2 * _H), jnp.float32),
        pltpu.VMEM((_K, 2 * _H), jnp.float32),
        pltpu.VMEM((_K, _D), jnp.float32),
        pltpu.VMEM((_K, _D), jnp.float32),
        pltpu.VMEM_SHARED((_NP, _D), jnp.float32),
        pltpu.VMEM_SHARED((_NP, 2 * _H), jnp.float32),
        pltpu.SemaphoreType.DMA,
        pltpu.SemaphoreType.DMA,
        pltpu.SemaphoreType.DMA,
    ],
    compiler_params=pltpu.CompilerParams(use_tc_tiling_on_sc=False,
                                         needs_layout_passes=False),
)(_sc_edge_body)


def kernel(x, edge_index, edge_attr, W_h, b_h, W_l, b_l, W_g, att_src, att_dst,
           g_in, b_in, g_n, b_n, b_param):
    rows = jnp.arange(_D, dtype=jnp.int32)
    ms = jnp.zeros((_D, _H), jnp.float32).at[rows, rows // _DH].set(att_src.reshape(-1))
    md = jnp.zeros((_D, _H), jnp.float32).at[rows, rows // _DH].set(att_dst.reshape(-1))
    rexp = jnp.zeros((_H, _D), jnp.float32).at[rows // _DH, rows].set(1.0)

    grid = _N // _BN
    row_spec = pl.BlockSpec((_BN, _D), lambda i: (i, 0))
    w_spec = pl.BlockSpec((_D, _D), lambda i: (0, 0))
    b_spec = pl.BlockSpec((1, _D), lambda i: (0, 0))
    m_spec = pl.BlockSpec((_D, _H), lambda i: (0, 0))
    a_spec = pl.BlockSpec((_BN, 2 * _H), lambda i: (i, 0))

    x0, h, xl, xp, atab1, atab2 = pl.pallas_call(
        _pre_body,
        grid=(grid,),
        in_specs=[row_spec, w_spec, b_spec, w_spec, b_spec, w_spec, m_spec,
                  m_spec, b_spec, b_spec],
        out_specs=[row_spec, row_spec, row_spec, row_spec, a_spec, a_spec],
        out_shape=[jax.ShapeDtypeStruct((_N, _D), jnp.float32)] * 4
        + [jax.ShapeDtypeStruct((_N, 2 * _H), jnp.float32)] * 2,
    )(x, W_h, b_h.reshape(1, _D), W_l, b_l.reshape(1, _D), W_g, ms, md,
      g_in.reshape(1, _D), b_in.reshape(1, _D))

    w2 = pl.pallas_call(
        _wexp_body,
        out_shape=jax.ShapeDtypeStruct((_E // _D, _D), jnp.float32),
    )(edge_attr.reshape(_E // _D, _D))

    pad = _EP - _E
    src = jnp.pad(edge_index[0].astype(jnp.int32), (0, pad)).reshape(_EP // _K, _K)
    dst = jnp.pad(edge_index[1].astype(jnp.int32), (0, pad)).reshape(_EP // _K, _K)
    w2p = jnp.broadcast_to(
        jnp.pad(w2.reshape(-1), (0, pad)).reshape(_EP // _K, _K, 1),
        (_EP // _K, _K, 16))
    z128 = jnp.zeros((_NP, _D), jnp.float32)
    z16 = jnp.zeros((_NP, 2 * _H), jnp.float32)

    acc, den = _sc_edge(src, dst, w2p, atab1, atab2, xp, z128, z16)

    d0 = den[0, :_N, :_H]
    d1 = den[1, :_N, :_H]
    d_spec = pl.BlockSpec((_BN, _H), lambda i: (i, 0))
    r_spec = pl.BlockSpec((_H, _D), lambda i: (0, 0))
    out = pl.pallas_call(
        _post_body,
        grid=(grid,),
        in_specs=[row_spec, row_spec, d_spec, d_spec, row_spec, row_spec,
                  row_spec, r_spec, b_spec, b_spec, b_spec],
        out_specs=row_spec,
        out_shape=jax.ShapeDtypeStruct((_N, _D), jnp.float32),
    )(acc[0, :_N], acc[1, :_N], d0, d1, x0, h, xl, rexp,
      g_n.reshape(1, _D), b_n.reshape(1, _D), b_param.reshape(1, _D))
    return out


# speedup vs baseline: 52.7157x; 1.2764x over previous
import functools

import jax
import jax.numpy as jnp
from jax import lax
from jax.experimental import pallas as pl
from jax.experimental.pallas import tpu as pltpu
from jax.experimental.pallas import tpu_sc as plsc

_N = 10000
_E = 320000
_D = 128
_H = 8
_DH = 16
_EPS = 1e-5
_NEG = 0.2
_INV_LN2 = 1.4426950408889634

_NC = 2
_NS = 16
_NW = _NC * _NS
_K = 64
_CPT = 160
_EP = _NW * _CPT * _K
_NP = 10240
_ROWS = _NP // _NS

_BN = 1000


def _pre_body(x_ref, wh_ref, bh_ref, wl_ref, bl_ref, wg_ref, ms_ref, md_ref,
              gin_ref, bin_ref, x0_ref, h_ref, xl_ref, xp_ref, a1_ref, a2_ref):
    x = x_ref[...]
    mu = jnp.mean(x, axis=1, keepdims=True)
    xc = x - mu
    var = jnp.mean(xc * xc, axis=1, keepdims=True)
    x0 = xc * lax.rsqrt(var + _EPS) * gin_ref[...] + bin_ref[...]
    x0_ref[...] = x0
    h_ref[...] = jnp.maximum(
        jnp.dot(x0, wh_ref[...], preferred_element_type=jnp.float32) + bh_ref[...], 0.0)
    xl_ref[...] = jnp.dot(x0, wl_ref[...], preferred_element_type=jnp.float32) + bl_ref[...]
    xp = jnp.dot(x0, wg_ref[...], preferred_element_type=jnp.float32)
    xp_ref[...] = xp
    a_s = jnp.dot(xp, ms_ref[...], preferred_element_type=jnp.float32)
    a_d = jnp.dot(xp, md_ref[...], preferred_element_type=jnp.float32)
    a1_ref[...] = jnp.concatenate([a_s, jnp.zeros_like(a_d)], axis=1)
    a2_ref[...] = jnp.concatenate([a_d, jnp.zeros_like(a_s)], axis=1)


def _wexp_body(w_ref, o_ref):
    o_ref[...] = jnp.exp(jnp.log(w_ref[...]) * _INV_LN2)


def _post_body(a0_ref, a1_ref, d0_ref, d1_ref, x0_ref, h_ref, xl_ref, r_ref,
               gn_ref, bn_ref, beta_ref, out_ref):
    den = d0_ref[...] + d1_ref[...] + 1e-16
    r128 = jnp.dot(1.0 / den, r_ref[...], preferred_element_type=jnp.float32)
    gat = (a0_ref[...] + a1_ref[...]) * r128
    xg = jnp.maximum(gat + xl_ref[...], 0.0)
    m = h_ref[...] * xg
    mu = jnp.mean(m, axis=1, keepdims=True)
    mc = m - mu
    var = jnp.mean(mc * mc, axis=1, keepdims=True)
    ln = mc * lax.rsqrt(var + _EPS) * gn_ref[...] + bn_ref[...]
    beta = beta_ref[...]
    out_ref[...] = (1.0 - beta) * ln + beta * xg + x0_ref[...]


def _sc_edge_body(sd_hbm, w_hbm, a1_hbm, a2_hbm, xp_hbm, z128_hbm,
                  z16_hbm, acc_hbm, den_hbm,
                  sd_v, w_v, as_v, ad_v, xp_v, e_v, msg_v,
                  acc_sh, den_sh, sem_i, sem_g):
    cid = lax.axis_index("c")
    sid = lax.axis_index("s")
    wid = cid * _NS + sid

    base = sid * _ROWS
    pltpu.sync_copy(z128_hbm.at[pl.ds(base, _ROWS)], acc_sh.at[pl.ds(base, _ROWS)])
    pltpu.sync_copy(z16_hbm.at[pl.ds(base, _ROWS)], den_sh.at[pl.ds(base, _ROWS)])
    plsc.subcore_barrier()

    def row_of(c):
        return wid * _CPT + c

    def issue_idx(c, r4):
        pltpu.async_copy(sd_hbm.at[row_of(c)], sd_v.at[r4], sem_i)
        pltpu.async_copy(w_hbm.at[row_of(c)], w_v.at[r4], sem_i)

    def wait_idx(r4):
        pltpu.make_async_copy(sd_hbm.at[0], sd_v.at[r4], sem_i).wait()
        pltpu.make_async_copy(w_hbm.at[0], w_v.at[r4], sem_i).wait()

    def issue_gathers(r4, r2):
        pltpu.async_copy(a1_hbm.at[sd_v.at[r4, 0]], as_v.at[r2], sem_g)
        pltpu.async_copy(a2_hbm.at[sd_v.at[r4, 1]], ad_v.at[r2], sem_g)
        pltpu.async_copy(xp_hbm.at[sd_v.at[r4, 0]], xp_v.at[r2], sem_g)

    def wait_gathers(r2):
        pltpu.make_async_copy(a1_hbm.at[sd_v.at[0, 0]], as_v.at[r2], sem_g).wait()
        pltpu.make_async_copy(a2_hbm.at[sd_v.at[0, 1]], ad_v.at[r2], sem_g).wait()
        pltpu.make_async_copy(xp_hbm.at[sd_v.at[0, 0]], xp_v.at[r2], sem_g).wait()

    pltpu.sync_copy(sd_hbm.at[row_of(0)], sd_v.at[0])
    pltpu.sync_copy(w_hbm.at[row_of(0)], w_v.at[0])
    issue_gathers(0, 0)
    issue_idx(1, 1)

    def quad_body(c4, _):
        for b4 in range(4):
            c = c4 * 4 + b4
            r2 = b4 & 1

            @pl.when(c + 1 < _CPT)
            def _():
                wait_idx((b4 + 1) % 4)
                issue_gathers((b4 + 1) % 4, 1 - r2)

            @pl.when(c + 2 < _CPT)
            def _():
                issue_idx(c + 2, (b4 + 2) % 4)

            wait_gathers(r2)

            @plsc.parallel_loop(0, _K, unroll=4)
            def edge_body(k):
                z = as_v[r2, k, :] + ad_v[r2, k, :]
                lr = jnp.where(z > 0.0, z, z * _NEG)
                e = jnp.exp(lr) * w_v[b4, k, :]
                e_v[k, :] = e
                for h in range(_H):
                    mult = jnp.full((16,), e[h], jnp.float32)
                    msg_v[k, pl.ds(h * 16, 16)] = xp_v[r2, k, pl.ds(h * 16, 16)] * mult

            pltpu.sync_copy(e_v, den_sh.at[sd_v.at[b4, 1]], add=True)
            pltpu.sync_copy(msg_v, acc_sh.at[sd_v.at[b4, 1]], add=True)
        return 0

    lax.fori_loop(0, _CPT // 4, quad_body, 0)
    plsc.subcore_barrier()

    pltpu.sync_copy(acc_sh.at[pl.ds(base, _ROWS)], acc_hbm.at[cid, pl.ds(base, _ROWS)])
    pltpu.sync_copy(den_sh.at[pl.ds(base, _ROWS)], den_hbm.at[cid, pl.ds(base, _ROWS)])


_sc_edge = functools.partial(
    pl.kernel,
    out_type=(jax.ShapeDtypeStruct((_NC, _NP, _D), jnp.float32),
              jax.ShapeDtypeStruct((_NC, _NP, 2 * _H), jnp.float32)),
    mesh=plsc.VectorSubcoreMesh(core_axis_name="c", subcore_axis_name="s"),
    scratch_types=[
        pltpu.VMEM((4, 2, _K), jnp.int32),
        pltpu.VMEM((4, _K, 16), jnp.float32),
        pltpu.VMEM((2, _K, 2 * _H), jnp.float32),
        pltpu.VMEM((2, _K, 2 * _H), jnp.float32),
        pltpu.VMEM((2, _K, _D), jnp.float32),
        pltpu.VMEM((_K, 2 * _H), jnp.float32),
        pltpu.VMEM((_K, _D), jnp.float32),
        pltpu.VMEM_SHARED((_NP, _D), jnp.float32),
        pltpu.VMEM_SHARED((_NP, 2 * _H), jnp.float32),
        pltpu.SemaphoreType.DMA,
        pltpu.SemaphoreType.DMA,
    ],
    compiler_params=pltpu.CompilerParams(use_tc_tiling_on_sc=False,
                                         needs_layout_passes=False),
)(_sc_edge_body)


def kernel(x, edge_index, edge_attr, W_h, b_h, W_l, b_l, W_g, att_src, att_dst,
           g_in, b_in, g_n, b_n, b_param):
    rows = jnp.arange(_D, dtype=jnp.int32)
    ms = jnp.zeros((_D, _H), jnp.float32).at[rows, rows // _DH].set(att_src.reshape(-1))
    md = jnp.zeros((_D, _H), jnp.float32).at[rows, rows // _DH].set(att_dst.reshape(-1))
    rexp = jnp.zeros((_H, _D), jnp.float32).at[rows // _DH, rows].set(1.0)

    grid = _N // _BN
    row_spec = pl.BlockSpec((_BN, _D), lambda i: (i, 0))
    w_spec = pl.BlockSpec((_D, _D), lambda i: (0, 0))
    b_spec = pl.BlockSpec((1, _D), lambda i: (0, 0))
    m_spec = pl.BlockSpec((_D, _H), lambda i: (0, 0))
    a_spec = pl.BlockSpec((_BN, 2 * _H), lambda i: (i, 0))

    x0, h, xl, xp, atab1, atab2 = pl.pallas_call(
        _pre_body,
        grid=(grid,),
        in_specs=[row_spec, w_spec, b_spec, w_spec, b_spec, w_spec, m_spec,
                  m_spec, b_spec, b_spec],
        out_specs=[row_spec, row_spec, row_spec, row_spec, a_spec, a_spec],
        out_shape=[jax.ShapeDtypeStruct((_N, _D), jnp.float32)] * 4
        + [jax.ShapeDtypeStruct((_N, 2 * _H), jnp.float32)] * 2,
    )(x, W_h, b_h.reshape(1, _D), W_l, b_l.reshape(1, _D), W_g, ms, md,
      g_in.reshape(1, _D), b_in.reshape(1, _D))

    w2 = pl.pallas_call(
        _wexp_body,
        out_shape=jax.ShapeDtypeStruct((_E // _D, _D), jnp.float32),
    )(edge_attr.reshape(_E // _D, _D))

    pad = _EP - _E
    sd = jnp.stack([
        jnp.pad(edge_index[0].astype(jnp.int32), (0, pad)).reshape(_EP // _K, _K),
        jnp.pad(edge_index[1].astype(jnp.int32), (0, pad)).reshape(_EP // _K, _K),
    ], axis=1)
    w2p = jnp.broadcast_to(
        jnp.pad(w2.reshape(-1), (0, pad)).reshape(_EP // _K, _K, 1),
        (_EP // _K, _K, 16))
    z128 = jnp.zeros((_NP, _D), jnp.float32)
    z16 = jnp.zeros((_NP, 2 * _H), jnp.float32)

    acc, den = _sc_edge(sd, w2p, atab1, atab2, xp, z128, z16)

    d0 = den[0, :_N, :_H]
    d1 = den[1, :_N, :_H]
    d_spec = pl.BlockSpec((_BN, _H), lambda i: (i, 0))
    r_spec = pl.BlockSpec((_H, _D), lambda i: (0, 0))
    out = pl.pallas_call(
        _post_body,
        grid=(grid,),
        in_specs=[row_spec, row_spec, d_spec, d_spec, row_spec, row_spec,
                  row_spec, r_spec, b_spec, b_spec, b_spec],
        out_specs=row_spec,
        out_shape=jax.ShapeDtypeStruct((_N, _D), jnp.float32),
    )(acc[0, :_N], acc[1, :_N], d0, d1, x0, h, xl, rexp,
      g_n.reshape(1, _D), b_n.reshape(1, _D), b_param.reshape(1, _D))
    return out
